# baseline (speedup 1.0000x reference)
import functools

import jax
import jax.numpy as jnp
from jax import lax
from jax.experimental import pallas as pl
from jax.experimental.pallas import tpu as pltpu
from jax.experimental.pallas import tpu_sc as plsc

VOCAB_ = 100000
D_ = 128
C_ = D_ // 2
B_ = 16384
K_ = 20
NW_ = 32
BPW_ = B_ // NW_
CH_ = 32
NCHUNK_ = BPW_ // CH_
NSUB_ = CH_ * K_ // 80
CSEGS_ = ((0, 8), (8, 16), (16, 32), (32, 64))


def _i32x16(v):
    return jnp.zeros((16,), jnp.int32) + v


def _rot(accs, j, t):
    out = list(accs)
    out[j % 4] = out[j % 4] + t
    return tuple(out)


def _ckpt(accs):
    m = (accs[0] + accs[1]) + (accs[2] + accs[3])
    w = plsc.bitcast(m, jnp.int32)
    lo = plsc.bitcast(w << 16, jnp.float32)
    hi = plsc.bitcast(w & (-65536), jnp.float32)
    return lo + hi


def _zaccs():
    return (jnp.zeros((32,), jnp.bfloat16),) * 4


def _sc_body(ctr_hbm, ctx_hbm, neg_hbm, win_hbm, wout_hbm,
             outa_hbm, outb_hbm, outc_hbm,
             ctr0, ctx0, neg0, rows0, vc0, vp0,
             ctr1, ctx1, neg1, rows1, vc1, vp1,
             oa0, ob0, oc0, oa1, ob1, oc1, sems):
    nc = plsc.get_sparse_core_info().num_cores
    wid = lax.axis_index("s") * nc + lax.axis_index("c")
    lanes = lax.iota(jnp.int32, 16)
    rsem0, rsem1 = sems.at[0], sems.at[1]
    isem0, isem1 = sems.at[2], sems.at[3]
    osem0, osem1 = sems.at[4], sems.at[5]

    def base_of(c):
        return pl.multiple_of(wid * BPW_ + c * CH_, CH_)

    def stage_idx(c, ctr_v, ctx_v, neg_v, isem):
        b0 = base_of(c)
        pltpu.make_async_copy(ctr_hbm.at[pl.ds(b0, CH_)], ctr_v, isem).start()
        pltpu.make_async_copy(ctx_hbm.at[pl.ds(b0, CH_)], ctx_v, isem).start()
        pltpu.make_async_copy(
            neg_hbm.at[pl.ds(b0 * K_, CH_ * K_)], neg_v, isem).start()

    def wait_idx(ctr_v, ctx_v, neg_v, isem):
        pltpu.make_async_copy(ctr_hbm.at[pl.ds(0, CH_)], ctr_v, isem).wait()
        pltpu.make_async_copy(ctx_hbm.at[pl.ds(0, CH_)], ctx_v, isem).wait()
        pltpu.make_async_copy(
            neg_hbm.at[pl.ds(0, CH_ * K_)], neg_v, isem).wait()

    def fire_rows(ctr_v, ctx_v, neg_v, rows_v, vc_v, vp_v, rsem):
        for j in range(NSUB_):
            sl = pl.ds(j * 80, 80)
            pltpu.make_async_copy(
                wout_hbm.at[neg_v.at[sl]], rows_v.at[sl], rsem).start()
        pltpu.make_async_copy(win_hbm.at[ctr_v], vc_v, rsem).start()
        pltpu.make_async_copy(wout_hbm.at[ctx_v], vp_v, rsem).start()

    def wait_rows(ctr_v, ctx_v, neg_v, rows_v, vc_v, vp_v, rsem):
        for j in range(NSUB_):
            sl = pl.ds(j * 80, 80)
            pltpu.make_async_copy(
                wout_hbm.at[neg_v.at[sl]], rows_v.at[sl], rsem).wait()
        pltpu.make_async_copy(win_hbm.at[ctr_v], vc_v, rsem).wait()
        pltpu.make_async_copy(wout_hbm.at[ctx_v], vp_v, rsem).wait()

    def fire_outs(c, oa_v, ob_v, oc_v, osem):
        b0 = base_of(c)
        pltpu.make_async_copy(
            oa_v, outa_hbm.at[pl.ds(b0, CH_)], osem).start()
        pltpu.make_async_copy(
            ob_v, outb_hbm.at[pl.ds(b0 // 4, CH_ // 4)], osem).start()
        pltpu.make_async_copy(
            oc_v, outc_hbm.at[pl.ds(b0 // 16, CH_ // 16)], osem).start()

    def wait_outs(oa_v, ob_v, oc_v, osem):
        pltpu.make_async_copy(oa_v, outa_hbm.at[pl.ds(0, CH_)], osem).wait()
        pltpu.make_async_copy(
            ob_v, outb_hbm.at[pl.ds(0, CH_ // 4)], osem).wait()
        pltpu.make_async_copy(
            oc_v, outc_hbm.at[pl.ds(0, CH_ // 16)], osem).wait()

    def compute(rows_v, vc_v, vp_v, oa_v, ob_v, oc_v):
        def pass1(b, _):
            rowv = b * K_ + lanes
            vcbs = [vc_v[b, pl.ds(o, 16)] for o in (0, 16, 32, 48)]
            accs = _zaccs()
            for mi, (lo, hi) in enumerate(CSEGS_):
                for c in range(lo, hi):
                    cw = vcbs[c // 16].at[jnp.full((16,), c % 16, jnp.int32)
                                          ].get(mode="promise_in_bounds")
                    cv = plsc.bitcast(cw, jnp.bfloat16)
                    vals = plsc.bitcast(
                        plsc.load_gather(rows_v, [rowv, _i32x16(c)]),
                        jnp.bfloat16)
                    accs = _rot(accs, c, vals * cv)
                oa_v[b, mi, :] = _ckpt(accs)
            return 0

        lax.fori_loop(0, CH_, pass1, 0)

        def pass2(sub, _):
            bvec = sub * 4 + lanes // 4
            rowv = bvec * K_ + 16 + (lanes % 4)
            accs = _zaccs()
            for mi, (lo, hi) in enumerate(CSEGS_):
                for c in range(lo, hi):
                    cols = _i32x16(c)
                    vals = plsc.bitcast(
                        plsc.load_gather(rows_v, [rowv, cols]), jnp.bfloat16)
                    cv = plsc.bitcast(
                        plsc.load_gather(vc_v, [bvec, cols]), jnp.bfloat16)
                    accs = _rot(accs, c, vals * cv)
                ob_v[sub, mi, :] = _ckpt(accs)
            return 0

        lax.fori_loop(0, CH_ // 4, pass2, 0)

        for g in range(CH_ // 16):
            bl = g * 16 + lanes
            accs = _zaccs()
            for mi, (lo, hi) in enumerate(CSEGS_):
                for c in range(lo, hi):
                    cols = _i32x16(c)
                    pv = plsc.bitcast(
                        plsc.load_gather(vp_v, [bl, cols]), jnp.bfloat16)
                    cv = plsc.bitcast(
                        plsc.load_gather(vc_v, [bl, cols]), jnp.bfloat16)
                    accs = _rot(accs, c, pv * cv)
                oc_v[g, mi, :] = _ckpt(accs)

    bufs0 = (ctr0, ctx0, neg0, rows0, vc0, vp0)
    bufs1 = (ctr1, ctx1, neg1, rows1, vc1, vp1)

    stage_idx(0, ctr0, ctx0, neg0, isem0)
    wait_idx(ctr0, ctx0, neg0, isem0)
    fire_rows(*bufs0, rsem0)
    stage_idx(1, ctr1, ctx1, neg1, isem1)

    def body(i, _):
        c0 = 2 * i
        wait_idx(ctr1, ctx1, neg1, isem1)
        fire_rows(*bufs1, rsem1)
        wait_rows(*bufs0, rsem0)

        @pl.when(i < NCHUNK_ // 2 - 1)
        def _():
            stage_idx(c0 + 2, ctr0, ctx0, neg0, isem0)

        @pl.when(i > 0)
        def _():
            wait_outs(oa0, ob0, oc0, osem0)

        compute(rows0, vc0, vp0, oa0, ob0, oc0)
        fire_outs(c0, oa0, ob0, oc0, osem0)

        @pl.when(i < NCHUNK_ // 2 - 1)
        def _():
            wait_idx(ctr0, ctx0, neg0, isem0)
            fire_rows(*bufs0, rsem0)

        wait_rows(*bufs1, rsem1)

        @pl.when(i < NCHUNK_ // 2 - 1)
        def _():
            stage_idx(c0 + 3, ctr1, ctx1, neg1, isem1)

        @pl.when(i > 0)
        def _():
            wait_outs(oa1, ob1, oc1, osem1)

        compute(rows1, vc1, vp1, oa1, ob1, oc1)
        fire_outs(c0 + 1, oa1, ob1, oc1, osem1)
        return 0

    lax.fori_loop(0, NCHUNK_ // 2, body, 0)
    wait_outs(oa0, ob0, oc0, osem0)
    wait_outs(oa1, ob1, oc1, osem1)


def _sc_scores(centers, contexts, negflat, win_pk, wout_pk):
    mesh = plsc.VectorSubcoreMesh(core_axis_name="c", subcore_axis_name="s")
    f32 = jnp.float32
    i32 = jnp.int32
    bufset = [
        pltpu.VMEM((CH_,), i32),
        pltpu.VMEM((CH_,), i32),
        pltpu.VMEM((CH_ * K_,), i32),
        pltpu.VMEM((CH_ * K_, C_), i32),
        pltpu.VMEM((CH_, C_), i32),
        pltpu.VMEM((CH_, C_), i32),
    ]
    outset = [
        pltpu.VMEM((CH_, 4, 16), f32),
        pltpu.VMEM((CH_ // 4, 4, 16), f32),
        pltpu.VMEM((CH_ // 16, 4, 16), f32),
    ]
    kern = functools.partial(
        pl.kernel,
        out_type=(
            jax.ShapeDtypeStruct((B_, 4, 16), f32),
            jax.ShapeDtypeStruct((B_ // 4, 4, 16), f32),
            jax.ShapeDtypeStruct((B_ // 16, 4, 16), f32),
        ),
        mesh=mesh,
        compiler_params=pltpu.CompilerParams(
            needs_layout_passes=False, use_tc_tiling_on_sc=False),
        scratch_types=bufset + bufset + outset + outset
        + [pltpu.SemaphoreType.DMA((6,))],
    )(_sc_body)
    return kern(centers, contexts, negflat, win_pk, wout_pk)


def _tc_reduce(nega, negb, posc):
    def body(a_ref, b_ref, c_ref, o_ref):
        s = jnp.sum(jax.nn.log_sigmoid(-a_ref[...]))
        s = s + jnp.sum(jax.nn.log_sigmoid(-b_ref[...]))
        s = s + jnp.sum(jax.nn.log_sigmoid(c_ref[...]))
        o_ref[...] = jnp.broadcast_to(-s * (0.25 / B_), (1, 1))

    return pl.pallas_call(
        body,
        out_shape=jax.ShapeDtypeStruct((1, 1), jnp.float32),
    )(nega, negb, posc)


def _pack(w):
    return lax.bitcast_convert_type(
        w.astype(jnp.bfloat16).reshape(VOCAB_, C_, 2), jnp.int32)


def kernel(centers, contexts, negatives, W_in, W_out):
    centers = centers.astype(jnp.int32)
    contexts = contexts.astype(jnp.int32)
    negflat = negatives.astype(jnp.int32).reshape(B_ * K_)
    nega, negb, posc = _sc_scores(
        centers, contexts, negflat, _pack(W_in), _pack(W_out))
    loss = _tc_reduce(
        nega.reshape(B_ * 64 // 128, 128),
        negb.reshape(B_ * 16 // 128, 128),
        posc.reshape(B_ * 4 // 128, 128),
    )
    return loss.reshape(())


# speedup vs baseline: 2.3875x; 2.3875x over previous
import functools

import jax
import jax.numpy as jnp
from jax import lax
from jax.experimental import pallas as pl
from jax.experimental.pallas import tpu as pltpu
from jax.experimental.pallas import tpu_sc as plsc

VOCAB_ = 100000
D_ = 128
C_ = D_ // 2
B_ = 16384
K_ = 20
NW_ = 32
BPW_ = B_ // NW_
CH_ = 16
NCHUNK_ = BPW_ // CH_
NSUB_ = CH_ * K_ // 80
CSEGS_ = ((0, 16), (16, 32), (32, 64))


def _i32x16(v):
    return jnp.zeros((16,), jnp.int32) + v


def _rot(accs, j, t):
    out = list(accs)
    out[j % 4] = out[j % 4] + t
    return tuple(out)


def _words(accs):
    m = (accs[0] + accs[1]) + (accs[2] + accs[3])
    return plsc.bitcast(m, jnp.int32)

def _lo(w):
    return plsc.bitcast(w << 16, jnp.float32)


def _hi(w):
    return plsc.bitcast(w & (-65536), jnp.float32)


def _zaccs():
    return (jnp.zeros((32,), jnp.bfloat16),) * 4


def _sc_body(ctr_hbm, ctx_hbm, neg_hbm, win_hbm, wout_hbm,
             outa_hbm, outb_hbm, outc_hbm,
             ctr0, ctx0, neg0, rows0, vc0, vp0,
             ctr1, ctx1, neg1, rows1, vc1, vp1,
             oa0, ob0, oc0, oa1, ob1, oc1, sems):
    nc = plsc.get_sparse_core_info().num_cores
    wid = lax.axis_index("s") * nc + lax.axis_index("c")
    lanes = lax.iota(jnp.int32, 16)
    rsem0, rsem1 = sems.at[0], sems.at[1]
    isem0, isem1 = sems.at[2], sems.at[3]
    osem0, osem1 = sems.at[4], sems.at[5]

    def base_of(c):
        return pl.multiple_of(wid * BPW_ + c * CH_, CH_)

    def stage_idx(c, ctr_v, ctx_v, neg_v, isem):
        b0 = base_of(c)
        pltpu.make_async_copy(ctr_hbm.at[pl.ds(b0, CH_)], ctr_v, isem).start()
        pltpu.make_async_copy(ctx_hbm.at[pl.ds(b0, CH_)], ctx_v, isem).start()
        pltpu.make_async_copy(
            neg_hbm.at[pl.ds(b0 * K_, CH_ * K_)], neg_v, isem).start()

    def wait_idx(ctr_v, ctx_v, neg_v, isem):
        pltpu.make_async_copy(ctr_hbm.at[pl.ds(0, CH_)], ctr_v, isem).wait()
        pltpu.make_async_copy(ctx_hbm.at[pl.ds(0, CH_)], ctx_v, isem).wait()
        pltpu.make_async_copy(
            neg_hbm.at[pl.ds(0, CH_ * K_)], neg_v, isem).wait()

    def fire_rows(ctr_v, ctx_v, neg_v, rows_v, vc_v, vp_v, rsem):
        for j in range(NSUB_):
            sl = pl.ds(j * 80, 80)
            pltpu.make_async_copy(
                wout_hbm.at[neg_v.at[sl]], rows_v.at[sl], rsem).start()
        pltpu.make_async_copy(win_hbm.at[ctr_v], vc_v, rsem).start()
        pltpu.make_async_copy(wout_hbm.at[ctx_v], vp_v, rsem).start()

    def wait_rows(ctr_v, ctx_v, neg_v, rows_v, vc_v, vp_v, rsem):
        for j in range(NSUB_):
            sl = pl.ds(j * 80, 80)
            pltpu.make_async_copy(
                wout_hbm.at[neg_v.at[sl]], rows_v.at[sl], rsem).wait()
        pltpu.make_async_copy(win_hbm.at[ctr_v], vc_v, rsem).wait()
        pltpu.make_async_copy(wout_hbm.at[ctx_v], vp_v, rsem).wait()

    def fire_outs(c, oa_v, ob_v, oc_v, osem):
        b0 = base_of(c)
        pltpu.make_async_copy(
            oa_v, outa_hbm.at[pl.ds(b0, CH_)], osem).start()
        pltpu.make_async_copy(
            ob_v, outb_hbm.at[pl.ds(b0 // 4, CH_ // 4)], osem).start()
        pltpu.make_async_copy(
            oc_v, outc_hbm.at[pl.ds(b0 // 16, CH_ // 16)], osem).start()

    def wait_outs(oa_v, ob_v, oc_v, osem):
        pltpu.make_async_copy(oa_v, outa_hbm.at[pl.ds(0, CH_)], osem).wait()
        pltpu.make_async_copy(
            ob_v, outb_hbm.at[pl.ds(0, CH_ // 4)], osem).wait()
        pltpu.make_async_copy(
            oc_v, outc_hbm.at[pl.ds(0, CH_ // 16)], osem).wait()

    def compute(rows_v, vc_v, vp_v, oa_v, ob_v, oc_v):
        def run_cols(acc_fn, store_fn):
            accs = _zaccs()
            for mi, (lo, hi) in enumerate(CSEGS_):
                for c in range(lo, hi):
                    accs = acc_fn(accs, c)
                w = _words(accs)
                store_fn(mi, _lo(w))
                if mi == 2:
                    store_fn(3, _lo(w) + _hi(w))

        @plsc.parallel_loop(0, CH_)
        def pass1(b):
            rowv = b * K_ + lanes
            vcbs = [vc_v[b, pl.ds(o, 16)] for o in (0, 16, 32, 48)]

            def acc_fn(accs, c):
                cw = vcbs[c // 16].at[jnp.full((16,), c % 16, jnp.int32)
                                      ].get(mode="promise_in_bounds")
                cv = plsc.bitcast(cw, jnp.bfloat16)
                vals = plsc.bitcast(
                    plsc.load_gather(rows_v, [rowv, _i32x16(c)]),
                    jnp.bfloat16)
                return _rot(accs, c, vals * cv)

            def store_fn(mi, v):
                oa_v[b, mi, :] = v

            run_cols(acc_fn, store_fn)

        @plsc.parallel_loop(0, CH_ // 4)
        def pass2(sub):
            bvec = sub * 4 + lanes // 4
            rowv = bvec * K_ + 16 + (lanes % 4)

            def acc_fn(accs, c):
                cols = _i32x16(c)
                vals = plsc.bitcast(
                    plsc.load_gather(rows_v, [rowv, cols]), jnp.bfloat16)
                cv = plsc.bitcast(
                    plsc.load_gather(vc_v, [bvec, cols]), jnp.bfloat16)
                return _rot(accs, c, vals * cv)

            def store_fn(mi, v):
                ob_v[sub, mi, :] = v

            run_cols(acc_fn, store_fn)

        for g in range(CH_ // 16):
            bl = g * 16 + lanes

            def acc_fn(accs, c):
                cols = _i32x16(c)
                pv = plsc.bitcast(
                    plsc.load_gather(vp_v, [bl, cols]), jnp.bfloat16)
                cv = plsc.bitcast(
                    plsc.load_gather(vc_v, [bl, cols]), jnp.bfloat16)
                return _rot(accs, c, pv * cv)

            def store_fn(mi, v, g=g):
                oc_v[g, mi, :] = v

            run_cols(acc_fn, store_fn)

    bufs0 = (ctr0, ctx0, neg0, rows0, vc0, vp0)
    bufs1 = (ctr1, ctx1, neg1, rows1, vc1, vp1)

    stage_idx(0, ctr0, ctx0, neg0, isem0)
    wait_idx(ctr0, ctx0, neg0, isem0)
    fire_rows(*bufs0, rsem0)
    stage_idx(1, ctr1, ctx1, neg1, isem1)

    def body(i, _):
        c0 = 2 * i
        wait_idx(ctr1, ctx1, neg1, isem1)
        fire_rows(*bufs1, rsem1)
        wait_rows(*bufs0, rsem0)

        @pl.when(i < NCHUNK_ // 2 - 1)
        def _():
            stage_idx(c0 + 2, ctr0, ctx0, neg0, isem0)

        @pl.when(i > 0)
        def _():
            wait_outs(oa0, ob0, oc0, osem0)

        compute(rows0, vc0, vp0, oa0, ob0, oc0)
        fire_outs(c0, oa0, ob0, oc0, osem0)

        @pl.when(i < NCHUNK_ // 2 - 1)
        def _():
            wait_idx(ctr0, ctx0, neg0, isem0)
            fire_rows(*bufs0, rsem0)

        wait_rows(*bufs1, rsem1)

        @pl.when(i < NCHUNK_ // 2 - 1)
        def _():
            stage_idx(c0 + 3, ctr1, ctx1, neg1, isem1)

        @pl.when(i > 0)
        def _():
            wait_outs(oa1, ob1, oc1, osem1)

        compute(rows1, vc1, vp1, oa1, ob1, oc1)
        fire_outs(c0 + 1, oa1, ob1, oc1, osem1)
        return 0

    lax.fori_loop(0, NCHUNK_ // 2, body, 0)
    wait_outs(oa0, ob0, oc0, osem0)
    wait_outs(oa1, ob1, oc1, osem1)


def _sc_scores(centers, contexts, negflat, win_pk, wout_pk):
    mesh = plsc.VectorSubcoreMesh(core_axis_name="c", subcore_axis_name="s")
    f32 = jnp.float32
    i32 = jnp.int32
    bufset = [
        pltpu.VMEM((CH_,), i32),
        pltpu.VMEM((CH_,), i32),
        pltpu.VMEM((CH_ * K_,), i32),
        pltpu.VMEM((CH_ * K_, D_), i32),
        pltpu.VMEM((CH_, D_), i32),
        pltpu.VMEM((CH_, D_), i32),
    ]
    outset = [
        pltpu.VMEM((CH_, 4, 16), f32),
        pltpu.VMEM((CH_ // 4, 4, 16), f32),
        pltpu.VMEM((CH_ // 16, 4, 16), f32),
    ]
    kern = functools.partial(
        pl.kernel,
        out_type=(
            jax.ShapeDtypeStruct((B_, 4, 16), f32),
            jax.ShapeDtypeStruct((B_ // 4, 4, 16), f32),
            jax.ShapeDtypeStruct((B_ // 16, 4, 16), f32),
        ),
        mesh=mesh,
        compiler_params=pltpu.CompilerParams(needs_layout_passes=False),
        scratch_types=bufset + bufset + outset + outset
        + [pltpu.SemaphoreType.DMA((6,))],
    )(_sc_body)
    return kern(centers, contexts, negflat, win_pk, wout_pk)


_PACK_ROWS_ = 2000


def _tc_pack(w):
    def body(w_ref, o_ref):
        u = lax.bitcast_convert_type(w_ref[...], jnp.uint32)
        one = jnp.uint32(1)
        r = (u + jnp.uint32(0x7FFF) + ((u >> 16) & one)) >> 16
        pk = r[:, :C_] | (r[:, C_:] << 16)
        pad = jnp.zeros_like(pk)
        o_ref[...] = lax.bitcast_convert_type(
            jnp.concatenate([pk, pad], axis=1), jnp.int32)

    return pl.pallas_call(
        body,
        grid=(VOCAB_ // _PACK_ROWS_,),
        in_specs=[pl.BlockSpec((_PACK_ROWS_, D_), lambda i: (i, 0))],
        out_specs=pl.BlockSpec((_PACK_ROWS_, D_), lambda i: (i, 0)),
        out_shape=jax.ShapeDtypeStruct((VOCAB_, D_), jnp.int32),
    )(w)


def _tc_reduce(nega, negb, posc):
    def body(a_ref, b_ref, c_ref, o_ref):
        s = jnp.sum(jax.nn.log_sigmoid(-a_ref[...]))
        s = s + jnp.sum(jax.nn.log_sigmoid(-b_ref[...]))
        s = s + jnp.sum(jax.nn.log_sigmoid(c_ref[...]))
        o_ref[...] = jnp.broadcast_to(-s * (0.25 / B_), (1, 1))

    return pl.pallas_call(
        body,
        out_shape=jax.ShapeDtypeStruct((1, 1), jnp.float32),
    )(nega, negb, posc)


def kernel(centers, contexts, negatives, W_in, W_out):
    centers = centers.astype(jnp.int32)
    contexts = contexts.astype(jnp.int32)
    negflat = negatives.astype(jnp.int32).reshape(B_ * K_)
    nega, negb, posc = _sc_scores(
        centers, contexts, negflat, _tc_pack(W_in), _tc_pack(W_out))
    loss = _tc_reduce(
        nega.reshape(B_ * 64 // 128, 128),
        negb.reshape(B_ * 16 // 128, 128),
        posc.reshape(B_ * 4 // 128, 128),
    )
    return loss.reshape(())


# speedup vs baseline: 4.3592x; 1.8259x over previous
import functools

import jax
import jax.numpy as jnp
from jax import lax
from jax.experimental import pallas as pl
from jax.experimental.pallas import tpu as pltpu
from jax.experimental.pallas import tpu_sc as plsc

VOCAB_ = 100000
D_ = 128
C_ = D_ // 2
B_ = 16384
K_ = 20
NW_ = 32
BPW_ = B_ // NW_
CH_ = 16
NCHUNK_ = BPW_ // CH_
NSUB_ = CH_ * K_ // 80
CSEGS_ = ((0, 16), (16, 32), (32, 64))


def _i32x16(v):
    return jnp.zeros((16,), jnp.int32) + v


def _rot(accs, j, t):
    out = list(accs)
    out[j % 4] = out[j % 4] + t
    return tuple(out)


def _words(accs):
    m = (accs[0] + accs[1]) + (accs[2] + accs[3])
    return plsc.bitcast(m, jnp.int32)

def _lo(w):
    return plsc.bitcast(w << 16, jnp.float32)


def _hi(w):
    return plsc.bitcast(w & (-65536), jnp.float32)


def _zaccs():
    return (jnp.zeros((32,), jnp.bfloat16),) * 4


def _sc_body(ctr_hbm, ctx_hbm, neg_hbm, win_hbm, wout_hbm,
             outa_hbm, outb_hbm, outc_hbm,
             ctr0, ctx0, neg0, rows0, vc0, vp0,
             ctr1, ctx1, neg1, rows1, vc1, vp1,
             oa0, ob0, oc0, oa1, ob1, oc1, colt_v, sems):
    nc = plsc.get_sparse_core_info().num_cores
    wid = lax.axis_index("s") * nc + lax.axis_index("c")
    lanes = lax.iota(jnp.int32, 16)
    rsem0, rsem1 = sems.at[0], sems.at[1]
    isem0, isem1 = sems.at[2], sems.at[3]
    osem0, osem1 = sems.at[4], sems.at[5]

    def base_of(c):
        return pl.multiple_of(wid * BPW_ + c * CH_, CH_)

    def stage_idx(c, ctr_v, ctx_v, neg_v, isem):
        b0 = base_of(c)
        pltpu.make_async_copy(ctr_hbm.at[pl.ds(b0, CH_)], ctr_v, isem).start()
        pltpu.make_async_copy(ctx_hbm.at[pl.ds(b0, CH_)], ctx_v, isem).start()
        pltpu.make_async_copy(
            neg_hbm.at[pl.ds(b0 * K_, CH_ * K_)], neg_v, isem).start()

    def wait_idx(ctr_v, ctx_v, neg_v, isem):
        pltpu.make_async_copy(ctr_hbm.at[pl.ds(0, CH_)], ctr_v, isem).wait()
        pltpu.make_async_copy(ctx_hbm.at[pl.ds(0, CH_)], ctx_v, isem).wait()
        pltpu.make_async_copy(
            neg_hbm.at[pl.ds(0, CH_ * K_)], neg_v, isem).wait()

    def fire_rows(ctr_v, ctx_v, neg_v, rows_v, vc_v, vp_v, rsem):
        for j in range(NSUB_):
            sl = pl.ds(j * 80, 80)
            pltpu.make_async_copy(
                wout_hbm.at[neg_v.at[sl]], rows_v.at[sl], rsem).start()
        pltpu.make_async_copy(win_hbm.at[ctr_v], vc_v, rsem).start()
        pltpu.make_async_copy(wout_hbm.at[ctx_v], vp_v, rsem).start()

    def wait_rows(ctr_v, ctx_v, neg_v, rows_v, vc_v, vp_v, rsem):
        for j in range(NSUB_):
            sl = pl.ds(j * 80, 80)
            pltpu.make_async_copy(
                wout_hbm.at[neg_v.at[sl]], rows_v.at[sl], rsem).wait()
        pltpu.make_async_copy(win_hbm.at[ctr_v], vc_v, rsem).wait()
        pltpu.make_async_copy(wout_hbm.at[ctx_v], vp_v, rsem).wait()

    def fire_outs(c, oa_v, ob_v, oc_v, osem):
        b0 = base_of(c)
        pltpu.make_async_copy(
            oa_v, outa_hbm.at[pl.ds(b0, CH_)], osem).start()
        pltpu.make_async_copy(
            ob_v, outb_hbm.at[pl.ds(b0 // 4, CH_ // 4)], osem).start()
        pltpu.make_async_copy(
            oc_v, outc_hbm.at[pl.ds(b0 // 16, CH_ // 16)], osem).start()

    def wait_outs(oa_v, ob_v, oc_v, osem):
        pltpu.make_async_copy(oa_v, outa_hbm.at[pl.ds(0, CH_)], osem).wait()
        pltpu.make_async_copy(
            ob_v, outb_hbm.at[pl.ds(0, CH_ // 4)], osem).wait()
        pltpu.make_async_copy(
            oc_v, outc_hbm.at[pl.ds(0, CH_ // 16)], osem).wait()

    def compute(rows_v, vc_v, vp_v, colt_v, oa_v, ob_v, oc_v):
        def run_cols(acc_fn, store_fn):
            accs = _zaccs()
            for mi, (lo, hi) in enumerate(CSEGS_):
                for s in range(lo, hi):
                    accs = acc_fn(accs, s, colt_v[s, :])
                w = _words(accs)
                store_fn(mi, _lo(w))
                if mi == 2:
                    store_fn(3, _lo(w) + _hi(w))

        @plsc.parallel_loop(0, CH_)
        def pass1(b):
            rowv = b * K_ + lanes
            bfull = jnp.zeros((16,), jnp.int32) + b

            def acc_fn(accs, s, colv):
                vals = plsc.bitcast(
                    plsc.load_gather(rows_v, [rowv, colv]), jnp.bfloat16)
                cv = plsc.bitcast(
                    plsc.load_gather(vc_v, [bfull, colv]), jnp.bfloat16)
                return _rot(accs, s, vals * cv)

            def store_fn(mi, v):
                oa_v[b, mi, :] = v

            run_cols(acc_fn, store_fn)

        @plsc.parallel_loop(0, CH_ // 4)
        def pass2(sub):
            bvec = sub * 4 + lanes // 4
            rowv = bvec * K_ + 16 + (lanes % 4)

            def acc_fn(accs, s, colv):
                vals = plsc.bitcast(
                    plsc.load_gather(rows_v, [rowv, colv]), jnp.bfloat16)
                cv = plsc.bitcast(
                    plsc.load_gather(vc_v, [bvec, colv]), jnp.bfloat16)
                return _rot(accs, s, vals * cv)

            def store_fn(mi, v):
                ob_v[sub, mi, :] = v

            run_cols(acc_fn, store_fn)

        for g in range(CH_ // 16):
            bl = g * 16 + lanes

            def acc_fn(accs, s, colv):
                pv = plsc.bitcast(
                    plsc.load_gather(vp_v, [bl, colv]), jnp.bfloat16)
                cv = plsc.bitcast(
                    plsc.load_gather(vc_v, [bl, colv]), jnp.bfloat16)
                return _rot(accs, s, pv * cv)

            def store_fn(mi, v, g=g):
                oc_v[g, mi, :] = v

            run_cols(acc_fn, store_fn)

    for mi, (lo, hi) in enumerate(CSEGS_):
        seglen = hi - lo
        for s in range(lo, hi):
            colt_v[s, :] = lo + ((lanes + (s - lo)) & (seglen - 1))

    bufs0 = (ctr0, ctx0, neg0, rows0, vc0, vp0)
    bufs1 = (ctr1, ctx1, neg1, rows1, vc1, vp1)

    stage_idx(0, ctr0, ctx0, neg0, isem0)
    wait_idx(ctr0, ctx0, neg0, isem0)
    fire_rows(*bufs0, rsem0)
    stage_idx(1, ctr1, ctx1, neg1, isem1)

    def body(i, _):
        c0 = 2 * i
        wait_idx(ctr1, ctx1, neg1, isem1)
        fire_rows(*bufs1, rsem1)
        wait_rows(*bufs0, rsem0)

        @pl.when(i < NCHUNK_ // 2 - 1)
        def _():
            stage_idx(c0 + 2, ctr0, ctx0, neg0, isem0)

        @pl.when(i > 0)
        def _():
            wait_outs(oa0, ob0, oc0, osem0)

        compute(rows0, vc0, vp0, colt_v, oa0, ob0, oc0)
        fire_outs(c0, oa0, ob0, oc0, osem0)

        @pl.when(i < NCHUNK_ // 2 - 1)
        def _():
            wait_idx(ctr0, ctx0, neg0, isem0)
            fire_rows(*bufs0, rsem0)

        wait_rows(*bufs1, rsem1)

        @pl.when(i < NCHUNK_ // 2 - 1)
        def _():
            stage_idx(c0 + 3, ctr1, ctx1, neg1, isem1)

        @pl.when(i > 0)
        def _():
            wait_outs(oa1, ob1, oc1, osem1)

        compute(rows1, vc1, vp1, colt_v, oa1, ob1, oc1)
        fire_outs(c0 + 1, oa1, ob1, oc1, osem1)
        return 0

    lax.fori_loop(0, NCHUNK_ // 2, body, 0)
    wait_outs(oa0, ob0, oc0, osem0)
    wait_outs(oa1, ob1, oc1, osem1)


def _sc_scores(centers, contexts, negflat, win_pk, wout_pk):
    mesh = plsc.VectorSubcoreMesh(core_axis_name="c", subcore_axis_name="s")
    f32 = jnp.float32
    i32 = jnp.int32
    bufset = [
        pltpu.VMEM((CH_,), i32),
        pltpu.VMEM((CH_,), i32),
        pltpu.VMEM((CH_ * K_,), i32),
        pltpu.VMEM((CH_ * K_, D_), i32),
        pltpu.VMEM((CH_, D_), i32),
        pltpu.VMEM((CH_, D_), i32),
    ]
    outset = [
        pltpu.VMEM((CH_, 4, 16), f32),
        pltpu.VMEM((CH_ // 4, 4, 16), f32),
        pltpu.VMEM((CH_ // 16, 4, 16), f32),
    ]
    kern = functools.partial(
        pl.kernel,
        out_type=(
            jax.ShapeDtypeStruct((B_, 4, 16), f32),
            jax.ShapeDtypeStruct((B_ // 4, 4, 16), f32),
            jax.ShapeDtypeStruct((B_ // 16, 4, 16), f32),
        ),
        mesh=mesh,
        compiler_params=pltpu.CompilerParams(needs_layout_passes=False),
        scratch_types=bufset + bufset + outset + outset
        + [pltpu.VMEM((64, 16), jnp.int32), pltpu.SemaphoreType.DMA((6,))],
    )(_sc_body)
    return kern(centers, contexts, negflat, win_pk, wout_pk)


_PACK_ROWS_ = 2000


def _tc_pack(w):
    def body(w_ref, o_ref):
        u = lax.bitcast_convert_type(w_ref[...], jnp.uint32)
        one = jnp.uint32(1)
        r = (u + jnp.uint32(0x7FFF) + ((u >> 16) & one)) >> 16
        pk = r[:, :C_] | (r[:, C_:] << 16)
        pad = jnp.zeros_like(pk)
        o_ref[...] = lax.bitcast_convert_type(
            jnp.concatenate([pk, pad], axis=1), jnp.int32)

    return pl.pallas_call(
        body,
        grid=(VOCAB_ // _PACK_ROWS_,),
        in_specs=[pl.BlockSpec((_PACK_ROWS_, D_), lambda i: (i, 0))],
        out_specs=pl.BlockSpec((_PACK_ROWS_, D_), lambda i: (i, 0)),
        out_shape=jax.ShapeDtypeStruct((VOCAB_, D_), jnp.int32),
    )(w)


def _tc_reduce(nega, negb, posc):
    def body(a_ref, b_ref, c_ref, o_ref):
        s = jnp.sum(jax.nn.log_sigmoid(-a_ref[...]))
        s = s + jnp.sum(jax.nn.log_sigmoid(-b_ref[...]))
        s = s + jnp.sum(jax.nn.log_sigmoid(c_ref[...]))
        o_ref[...] = jnp.broadcast_to(-s * (0.25 / B_), (1, 1))

    return pl.pallas_call(
        body,
        out_shape=jax.ShapeDtypeStruct((1, 1), jnp.float32),
    )(nega, negb, posc)


def kernel(centers, contexts, negatives, W_in, W_out):
    centers = centers.astype(jnp.int32)
    contexts = contexts.astype(jnp.int32)
    negflat = negatives.astype(jnp.int32).reshape(B_ * K_)
    nega, negb, posc = _sc_scores(
        centers, contexts, negflat, _tc_pack(W_in), _tc_pack(W_out))
    loss = _tc_reduce(
        nega.reshape(B_ * 64 // 128, 128),
        negb.reshape(B_ * 16 // 128, 128),
        posc.reshape(B_ * 4 // 128, 128),
    )
    return loss.reshape(())


# speedup vs baseline: 5.0725x; 1.1636x over previous
import functools

import jax
import jax.numpy as jnp
from jax import lax
from jax.experimental import pallas as pl
from jax.experimental.pallas import tpu as pltpu
from jax.experimental.pallas import tpu_sc as plsc

VOCAB_ = 100000
D_ = 128
B_ = 16384
K_ = 20
NW_ = 32
BPW_ = B_ // NW_
CH_ = 16
NCHUNK_ = BPW_ // CH_
NSUB_ = CH_ * K_ // 80
DSEGS_ = ((0, 16), (16, 32), (32, 64), (64, 128))


def _rot(accs, j, t):
    out = list(accs)
    out[j % 4] = out[j % 4] + t
    return tuple(out)


def _merge(accs):
    return (accs[0] + accs[1]) + (accs[2] + accs[3])


def _zaccs():
    return (jnp.zeros((16,), jnp.float32),) * 4


def _sc_body(ctr_hbm, ctx_hbm, neg_hbm, win_hbm, wout_hbm,
             outa_hbm, outb_hbm, outc_hbm,
             ctr0, ctx0, neg0, rows0, vc0, vp0,
             ctr1, ctx1, neg1, rows1, vc1, vp1,
             oa0, ob0, oc0, oa1, ob1, oc1, colt_v, sems):
    nc = plsc.get_sparse_core_info().num_cores
    wid = lax.axis_index("s") * nc + lax.axis_index("c")
    lanes = lax.iota(jnp.int32, 16)
    rsem0, rsem1 = sems.at[0], sems.at[1]
    isem0, isem1 = sems.at[2], sems.at[3]
    osem0, osem1 = sems.at[4], sems.at[5]

    def base_of(c):
        return pl.multiple_of(wid * BPW_ + c * CH_, CH_)

    def stage_idx(c, ctr_v, ctx_v, neg_v, isem):
        b0 = base_of(c)
        pltpu.make_async_copy(ctr_hbm.at[pl.ds(b0, CH_)], ctr_v, isem).start()
        pltpu.make_async_copy(ctx_hbm.at[pl.ds(b0, CH_)], ctx_v, isem).start()
        pltpu.make_async_copy(
            neg_hbm.at[pl.ds(b0 * K_, CH_ * K_)], neg_v, isem).start()

    def wait_idx(ctr_v, ctx_v, neg_v, isem):
        pltpu.make_async_copy(ctr_hbm.at[pl.ds(0, CH_)], ctr_v, isem).wait()
        pltpu.make_async_copy(ctx_hbm.at[pl.ds(0, CH_)], ctx_v, isem).wait()
        pltpu.make_async_copy(
            neg_hbm.at[pl.ds(0, CH_ * K_)], neg_v, isem).wait()

    def fire_rows(ctr_v, ctx_v, neg_v, rows_v, vc_v, vp_v, rsem):
        for j in range(NSUB_):
            sl = pl.ds(j * 80, 80)
            pltpu.make_async_copy(
                wout_hbm.at[neg_v.at[sl]], rows_v.at[sl], rsem).start()
        pltpu.make_async_copy(win_hbm.at[ctr_v], vc_v, rsem).start()
        pltpu.make_async_copy(wout_hbm.at[ctx_v], vp_v, rsem).start()

    def wait_rows(ctr_v, ctx_v, neg_v, rows_v, vc_v, vp_v, rsem):
        for j in range(NSUB_):
            sl = pl.ds(j * 80, 80)
            pltpu.make_async_copy(
                wout_hbm.at[neg_v.at[sl]], rows_v.at[sl], rsem).wait()
        pltpu.make_async_copy(win_hbm.at[ctr_v], vc_v, rsem).wait()
        pltpu.make_async_copy(wout_hbm.at[ctx_v], vp_v, rsem).wait()

    def fire_outs(c, oa_v, ob_v, oc_v, osem):
        b0 = base_of(c)
        pltpu.make_async_copy(
            oa_v, outa_hbm.at[pl.ds(b0, CH_)], osem).start()
        pltpu.make_async_copy(
            ob_v, outb_hbm.at[pl.ds(b0 // 4, CH_ // 4)], osem).start()
        pltpu.make_async_copy(
            oc_v, outc_hbm.at[pl.ds(b0 // 16, CH_ // 16)], osem).start()

    def wait_outs(oa_v, ob_v, oc_v, osem):
        pltpu.make_async_copy(oa_v, outa_hbm.at[pl.ds(0, CH_)], osem).wait()
        pltpu.make_async_copy(
            ob_v, outb_hbm.at[pl.ds(0, CH_ // 4)], osem).wait()
        pltpu.make_async_copy(
            oc_v, outc_hbm.at[pl.ds(0, CH_ // 16)], osem).wait()

    def compute(rows_v, vc_v, vp_v, oa_v, ob_v, oc_v):
        def run_dims(acc_fn, store_fn):
            accs = _zaccs()
            for mi, (lo, hi) in enumerate(DSEGS_):
                for s in range(lo, hi):
                    rotv = colt_v[s % 16, :]
                    accs = acc_fn(accs, s, rotv + (s & ~15), rotv)
                store_fn(mi, _merge(accs))

        @plsc.parallel_loop(0, CH_)
        def pass1(b):
            rowv = b * K_ + lanes
            vcbs = [vc_v[b, pl.ds(o, 16)] for o in range(0, D_, 16)]

            def acc_fn(accs, s, colv, rotv):
                vals = plsc.load_gather(rows_v, [rowv, colv])
                cv = vcbs[s // 16].at[rotv].get(mode="promise_in_bounds")
                return _rot(accs, s, vals * cv)

            def store_fn(mi, v):
                oa_v[b, mi, :] = v

            run_dims(acc_fn, store_fn)

        @plsc.parallel_loop(0, CH_ // 4)
        def pass2(sub):
            bvec = sub * 4 + lanes // 4
            rowv = bvec * K_ + 16 + (lanes % 4)

            def acc_fn(accs, s, colv, rotv):
                vals = plsc.load_gather(rows_v, [rowv, colv])
                cv = plsc.load_gather(vc_v, [bvec, colv])
                return _rot(accs, s, vals * cv)

            def store_fn(mi, v):
                ob_v[sub, mi, :] = v

            run_dims(acc_fn, store_fn)

        for g in range(CH_ // 16):
            bl = g * 16 + lanes

            def acc_fn(accs, s, colv, rotv):
                pv = plsc.load_gather(vp_v, [bl, colv])
                cv = plsc.load_gather(vc_v, [bl, colv])
                return _rot(accs, s, pv * cv)

            def store_fn(mi, v, g=g):
                oc_v[g, mi, :] = v

            run_dims(acc_fn, store_fn)

    for r in range(16):
        colt_v[r, :] = (lanes + r) & 15

    bufs0 = (ctr0, ctx0, neg0, rows0, vc0, vp0)
    bufs1 = (ctr1, ctx1, neg1, rows1, vc1, vp1)

    stage_idx(0, ctr0, ctx0, neg0, isem0)
    wait_idx(ctr0, ctx0, neg0, isem0)
    fire_rows(*bufs0, rsem0)
    stage_idx(1, ctr1, ctx1, neg1, isem1)

    def body(i, _):
        c0 = 2 * i
        wait_idx(ctr1, ctx1, neg1, isem1)
        fire_rows(*bufs1, rsem1)
        wait_rows(*bufs0, rsem0)

        @pl.when(i < NCHUNK_ // 2 - 1)
        def _():
            stage_idx(c0 + 2, ctr0, ctx0, neg0, isem0)

        @pl.when(i > 0)
        def _():
            wait_outs(oa0, ob0, oc0, osem0)

        compute(rows0, vc0, vp0, oa0, ob0, oc0)
        fire_outs(c0, oa0, ob0, oc0, osem0)

        @pl.when(i < NCHUNK_ // 2 - 1)
        def _():
            wait_idx(ctr0, ctx0, neg0, isem0)
            fire_rows(*bufs0, rsem0)

        wait_rows(*bufs1, rsem1)

        @pl.when(i < NCHUNK_ // 2 - 1)
        def _():
            stage_idx(c0 + 3, ctr1, ctx1, neg1, isem1)

        @pl.when(i > 0)
        def _():
            wait_outs(oa1, ob1, oc1, osem1)

        compute(rows1, vc1, vp1, oa1, ob1, oc1)
        fire_outs(c0 + 1, oa1, ob1, oc1, osem1)
        return 0

    lax.fori_loop(0, NCHUNK_ // 2, body, 0)
    wait_outs(oa0, ob0, oc0, osem0)
    wait_outs(oa1, ob1, oc1, osem1)


def _sc_scores(centers, contexts, negflat, w_in, w_out):
    mesh = plsc.VectorSubcoreMesh(core_axis_name="c", subcore_axis_name="s")
    f32 = jnp.float32
    i32 = jnp.int32
    bufset = [
        pltpu.VMEM((CH_,), i32),
        pltpu.VMEM((CH_,), i32),
        pltpu.VMEM((CH_ * K_,), i32),
        pltpu.VMEM((CH_ * K_, D_), f32),
        pltpu.VMEM((CH_, D_), f32),
        pltpu.VMEM((CH_, D_), f32),
    ]
    outset = [
        pltpu.VMEM((CH_, 4, 16), f32),
        pltpu.VMEM((CH_ // 4, 4, 16), f32),
        pltpu.VMEM((CH_ // 16, 4, 16), f32),
    ]
    kern = functools.partial(
        pl.kernel,
        out_type=(
            jax.ShapeDtypeStruct((B_, 4, 16), f32),
            jax.ShapeDtypeStruct((B_ // 4, 4, 16), f32),
            jax.ShapeDtypeStruct((B_ // 16, 4, 16), f32),
        ),
        mesh=mesh,
        compiler_params=pltpu.CompilerParams(needs_layout_passes=False),
        scratch_types=bufset + bufset + outset + outset
        + [pltpu.VMEM((16, 16), i32), pltpu.SemaphoreType.DMA((6,))],
    )(_sc_body)
    return kern(centers, contexts, negflat, w_in, w_out)


def _tc_reduce(nega, negb, posc):
    def body(a_ref, b_ref, c_ref, o_ref):
        s = jnp.sum(jax.nn.log_sigmoid(-a_ref[...]))
        s = s + jnp.sum(jax.nn.log_sigmoid(-b_ref[...]))
        s = s + jnp.sum(jax.nn.log_sigmoid(c_ref[...]))
        o_ref[...] = jnp.broadcast_to(-s * (0.25 / B_), (1, 1))

    return pl.pallas_call(
        body,
        out_shape=jax.ShapeDtypeStruct((1, 1), jnp.float32),
    )(nega, negb, posc)


def kernel(centers, contexts, negatives, W_in, W_out):
    centers = centers.astype(jnp.int32)
    contexts = contexts.astype(jnp.int32)
    negflat = negatives.astype(jnp.int32).reshape(B_ * K_)
    nega, negb, posc = _sc_scores(centers, contexts, negflat, W_in, W_out)
    loss = _tc_reduce(
        nega.reshape(B_ * 64 // 128, 128),
        negb.reshape(B_ * 16 // 128, 128),
        posc.reshape(B_ * 4 // 128, 128),
    )
    return loss.reshape(())
